# baseline (speedup 1.0000x reference)
import functools

import jax
import jax.numpy as jnp
from jax import lax
from jax.experimental import pallas as pl
from jax.experimental.pallas import tpu as pltpu
from jax.experimental.pallas import tpu_sc as plsc

N = 10000
E = 320000
D = 128
H = 128

NC = 2
NS = 16
NW = NC * NS
EPW = E // NW
K = 80
CHUNKS = EPW // K
NP = 10240
RPT = NP // NS
DEGW = 16
ROWW = D + DEGW


def _sc_body(src_hbm, dst_hbm, h_hbm, zacc_hbm,
             acc_out,
             si0, si1, si2, si3, di0, di1, di2, di3, rows0, rows1,
             acc_s,
             isem0, isem1, isem2, isem3, gsem0, gsem1):
  cid = lax.axis_index("c")
  sid = lax.axis_index("s")
  wid = sid * NC + cid
  ebase = pl.multiple_of(wid * EPW, 8)
  rbase = pl.multiple_of(sid * RPT, 8)

  pltpu.sync_copy(zacc_hbm.at[pl.ds(rbase, RPT)], acc_s.at[pl.ds(rbase, RPT)])
  plsc.subcore_barrier()

  sis = (si0, si1, si2, si3)
  dis = (di0, di1, di2, di3)
  rows = (rows0, rows1)
  isems = (isem0, isem1, isem2, isem3)
  gsems = (gsem0, gsem1)

  def issue_idx(g, s):
    base = pl.multiple_of(ebase + g * K, 8)
    pltpu.async_copy(src_hbm.at[pl.ds(base, K)], sis[s], isems[s])
    pltpu.async_copy(dst_hbm.at[pl.ds(base, K)], dis[s], isems[s])

  def issue_gather(g, s, r):
    base = pl.multiple_of(ebase + g * K, 8)
    pltpu.make_async_copy(src_hbm.at[pl.ds(base, K)], sis[s], isems[s]).wait()
    pltpu.make_async_copy(dst_hbm.at[pl.ds(base, K)], dis[s], isems[s]).wait()
    pltpu.async_copy(h_hbm.at[sis[s]], rows[r], gsems[r])

  def finish(g, s, r):
    pltpu.make_async_copy(h_hbm.at[sis[s]], rows[r], gsems[r]).wait()
    pltpu.sync_copy(rows[r], acc_s.at[dis[s]], add=True)

  issue_idx(0, 0)
  issue_idx(1, 1)
  issue_idx(2, 2)
  issue_idx(3, 3)
  issue_gather(0, 0, 0)

  @pl.loop(0, CHUNKS - 7, step=4)
  def _(i):
    issue_gather(i + 1, 1, 1)
    finish(i, 0, 0)
    issue_idx(i + 4, 0)
    issue_gather(i + 2, 2, 0)
    finish(i + 1, 1, 1)
    issue_idx(i + 5, 1)
    issue_gather(i + 3, 3, 1)
    finish(i + 2, 2, 0)
    issue_idx(i + 6, 2)
    issue_gather(i + 4, 0, 0)
    finish(i + 3, 3, 1)
    issue_idx(i + 7, 3)

  issue_gather(CHUNKS - 4, 1, 1)
  finish(CHUNKS - 5, 0, 0)
  issue_idx(CHUNKS - 1, 0)
  issue_gather(CHUNKS - 3, 2, 0)
  finish(CHUNKS - 4, 1, 1)
  issue_gather(CHUNKS - 2, 3, 1)
  finish(CHUNKS - 3, 2, 0)
  issue_gather(CHUNKS - 1, 0, 0)
  finish(CHUNKS - 2, 3, 1)
  finish(CHUNKS - 1, 0, 0)

  plsc.subcore_barrier()
  pltpu.sync_copy(acc_s.at[pl.ds(rbase, RPT)],
                  acc_out.at[cid, pl.ds(rbase, RPT)])


@functools.lru_cache(maxsize=1)
def _sc_aggregate():
  return pl.kernel(
    _sc_body,
    out_type=jax.ShapeDtypeStruct((NC, NP, ROWW), jnp.float32),
    mesh=plsc.VectorSubcoreMesh(
        core_axis_name="c", subcore_axis_name="s",
        num_cores=NC, num_subcores=NS),
    scratch_types=(
        [pltpu.VMEM((K,), jnp.int32)] * 8
        + [pltpu.VMEM((K, ROWW), jnp.float32)] * 2
        + [pltpu.VMEM_SHARED((NP, ROWW), jnp.float32)]
        + [pltpu.SemaphoreType.DMA] * 6
    ),
    compiler_params=pltpu.CompilerParams(use_tc_tiling_on_sc=False),
  )


BN = 400


def _tc_body(h_ref, acc_ref, w1h_ref, w1m_ref, b1_ref, w2_ref,
             b2_ref, out_ref):
  hh = h_ref[...]
  acc = acc_ref[0] + acc_ref[1]
  agg = acc[:, :D]
  deg = acc[:, D:D + 1]
  mean = agg / jnp.maximum(deg, 1.0)
  mean = jnp.where(deg == 0.0, hh, mean)
  x = (lax.dot_general(hh, w1h_ref[...], (((1,), (1,)), ((), ())),
                       preferred_element_type=jnp.float32)
       + lax.dot_general(mean, w1m_ref[...], (((1,), (1,)), ((), ())),
                         preferred_element_type=jnp.float32)
       + b1_ref[...])
  x = 0.5 * x * (1.0 + lax.erf(x * (2.0 ** -0.5)))
  out_ref[...] = (lax.dot_general(x, w2_ref[...], (((1,), (1,)), ((), ())),
                                  preferred_element_type=jnp.float32)
                  + b2_ref[...])


def _tc_mlp(h, acc2, w1h, w1m, b1, w2, b2):
  return pl.pallas_call(
      _tc_body,
      grid=(N // BN,),
      in_specs=[
          pl.BlockSpec((BN, D), lambda i: (i, 0)),
          pl.BlockSpec((NC, BN, ROWW), lambda i: (0, i, 0)),
          pl.BlockSpec((H, D), lambda i: (0, 0)),
          pl.BlockSpec((H, D), lambda i: (0, 0)),
          pl.BlockSpec((1, H), lambda i: (0, 0)),
          pl.BlockSpec((D, H), lambda i: (0, 0)),
          pl.BlockSpec((1, D), lambda i: (0, 0)),
      ],
      out_specs=pl.BlockSpec((BN, D), lambda i: (i, 0)),
      out_shape=jax.ShapeDtypeStruct((N, D), jnp.float32),
      compiler_params=pltpu.CompilerParams(
          dimension_semantics=("parallel",)),
  )(h, acc2, w1h, w1m, b1, w2, b2)


@jax.jit
def kernel(h, edge_index, W1, b1, W2, b2):
  src = edge_index[0]
  dst = edge_index[1]
  h_ext = jnp.concatenate([h, jnp.ones((N, DEGW), jnp.float32)], axis=1)
  zacc = jnp.zeros((NP, ROWW), jnp.float32)
  acc2 = _sc_aggregate()(src, dst, h_ext, zacc)
  return _tc_mlp(h, acc2, W1[:, :D], W1[:, D:],
                 b1.reshape(1, H), W2, b2.reshape(1, D))


# speedup vs baseline: 1.1724x; 1.1724x over previous
import functools

import jax
import jax.numpy as jnp
from jax import lax
from jax.experimental import pallas as pl
from jax.experimental.pallas import tpu as pltpu
from jax.experimental.pallas import tpu_sc as plsc

N = 10000
E = 320000
D = 128
H = 128

NC = 2
NS = 16
NW = NC * NS
EPW = E // NW
K = 80
CHUNKS = EPW // K
NP = 10240
RPT = NP // NS
DEGW = 16


def _sc_body(src_hbm, dst_hbm, h_hbm, zagg_hbm, zdeg_hbm,
             agg_out, deg_out,
             si0, si1, si2, si3, di0, di1, di2, di3, rows0, rows1,
             ones_v, agg_s, deg_s,
             isem0, isem1, isem2, isem3, gsem0, gsem1, ssem):
  cid = lax.axis_index("c")
  sid = lax.axis_index("s")
  wid = sid * NC + cid
  ebase = pl.multiple_of(wid * EPW, 8)
  rbase = pl.multiple_of(sid * RPT, 8)

  pltpu.sync_copy(zagg_hbm.at[pl.ds(rbase, RPT)], agg_s.at[pl.ds(rbase, RPT)])
  pltpu.sync_copy(zdeg_hbm.at[pl.ds(rbase, RPT)], deg_s.at[pl.ds(rbase, RPT)])
  for r in range(K):
    ones_v[r, :] = jnp.full((DEGW,), 1.0, jnp.float32)
  plsc.subcore_barrier()

  sis = (si0, si1, si2, si3)
  dis = (di0, di1, di2, di3)
  rows = (rows0, rows1)
  isems = (isem0, isem1, isem2, isem3)
  gsems = (gsem0, gsem1)

  def issue_idx(g, s):
    base = pl.multiple_of(ebase + g * K, 8)
    pltpu.async_copy(src_hbm.at[pl.ds(base, K)], sis[s], isems[s])
    pltpu.async_copy(dst_hbm.at[pl.ds(base, K)], dis[s], isems[s])

  def issue_gather(g, s, r):
    base = pl.multiple_of(ebase + g * K, 8)
    pltpu.make_async_copy(src_hbm.at[pl.ds(base, K)], sis[s], isems[s]).wait()
    pltpu.make_async_copy(dst_hbm.at[pl.ds(base, K)], dis[s], isems[s]).wait()
    pltpu.async_copy(h_hbm.at[sis[s]], rows[r], gsems[r])

  def finish(g, s, r):
    pltpu.make_async_copy(h_hbm.at[sis[s]], rows[r], gsems[r]).wait()
    pltpu.async_copy(rows[r], agg_s.at[dis[s]], ssem, add=True)
    pltpu.async_copy(ones_v, deg_s.at[dis[s]], ssem, add=True)
    pltpu.make_async_copy(rows[r], agg_s.at[dis[s]], ssem).wait()
    pltpu.make_async_copy(ones_v, deg_s.at[dis[s]], ssem).wait()

  issue_idx(0, 0)
  issue_idx(1, 1)
  issue_idx(2, 2)
  issue_idx(3, 3)
  issue_gather(0, 0, 0)

  @pl.loop(0, CHUNKS - 7, step=4)
  def _(i):
    issue_gather(i + 1, 1, 1)
    finish(i, 0, 0)
    issue_idx(i + 4, 0)
    issue_gather(i + 2, 2, 0)
    finish(i + 1, 1, 1)
    issue_idx(i + 5, 1)
    issue_gather(i + 3, 3, 1)
    finish(i + 2, 2, 0)
    issue_idx(i + 6, 2)
    issue_gather(i + 4, 0, 0)
    finish(i + 3, 3, 1)
    issue_idx(i + 7, 3)

  issue_gather(CHUNKS - 4, 1, 1)
  finish(CHUNKS - 5, 0, 0)
  issue_idx(CHUNKS - 1, 0)
  issue_gather(CHUNKS - 3, 2, 0)
  finish(CHUNKS - 4, 1, 1)
  issue_gather(CHUNKS - 2, 3, 1)
  finish(CHUNKS - 3, 2, 0)
  issue_gather(CHUNKS - 1, 0, 0)
  finish(CHUNKS - 2, 3, 1)
  finish(CHUNKS - 1, 0, 0)

  plsc.subcore_barrier()
  pltpu.sync_copy(agg_s.at[pl.ds(rbase, RPT)],
                  agg_out.at[cid, pl.ds(rbase, RPT)])
  pltpu.sync_copy(deg_s.at[pl.ds(rbase, RPT)],
                  deg_out.at[cid, pl.ds(rbase, RPT)])


@functools.lru_cache(maxsize=1)
def _sc_aggregate():
  return pl.kernel(
    _sc_body,
    out_type=[
        jax.ShapeDtypeStruct((NC, NP, D), jnp.float32),
        jax.ShapeDtypeStruct((NC, NP, DEGW), jnp.float32),
    ],
    mesh=plsc.VectorSubcoreMesh(
        core_axis_name="c", subcore_axis_name="s",
        num_cores=NC, num_subcores=NS),
    scratch_types=(
        [pltpu.VMEM((K,), jnp.int32)] * 8
        + [pltpu.VMEM((K, D), jnp.float32)] * 2
        + [
            pltpu.VMEM((K, DEGW), jnp.float32),
            pltpu.VMEM_SHARED((NP, D), jnp.float32),
            pltpu.VMEM_SHARED((NP, DEGW), jnp.float32),
        ]
        + [pltpu.SemaphoreType.DMA] * 7
    ),
    compiler_params=pltpu.CompilerParams(use_tc_tiling_on_sc=False),
  )


BN = 400


def _tc_body(h_ref, agg_ref, deg_ref, w1h_ref, w1m_ref, b1_ref, w2_ref,
             b2_ref, out_ref):
  hh = h_ref[...]
  agg = agg_ref[0] + agg_ref[1]
  deg = (deg_ref[0] + deg_ref[1])[:, 0:1]
  mean = agg / jnp.maximum(deg, 1.0)
  mean = jnp.where(deg == 0.0, hh, mean)
  x = (lax.dot_general(hh, w1h_ref[...], (((1,), (1,)), ((), ())),
                       preferred_element_type=jnp.float32)
       + lax.dot_general(mean, w1m_ref[...], (((1,), (1,)), ((), ())),
                         preferred_element_type=jnp.float32)
       + b1_ref[...])
  x = 0.5 * x * (1.0 + lax.erf(x * (2.0 ** -0.5)))
  out_ref[...] = (lax.dot_general(x, w2_ref[...], (((1,), (1,)), ((), ())),
                                  preferred_element_type=jnp.float32)
                  + b2_ref[...])


def _tc_mlp(h, agg2, deg2, w1h, w1m, b1, w2, b2):
  return pl.pallas_call(
      _tc_body,
      grid=(N // BN,),
      in_specs=[
          pl.BlockSpec((BN, D), lambda i: (i, 0)),
          pl.BlockSpec((NC, BN, D), lambda i: (0, i, 0)),
          pl.BlockSpec((NC, BN, DEGW), lambda i: (0, i, 0)),
          pl.BlockSpec((H, D), lambda i: (0, 0)),
          pl.BlockSpec((H, D), lambda i: (0, 0)),
          pl.BlockSpec((1, H), lambda i: (0, 0)),
          pl.BlockSpec((D, H), lambda i: (0, 0)),
          pl.BlockSpec((1, D), lambda i: (0, 0)),
      ],
      out_specs=pl.BlockSpec((BN, D), lambda i: (i, 0)),
      out_shape=jax.ShapeDtypeStruct((N, D), jnp.float32),
      compiler_params=pltpu.CompilerParams(
          dimension_semantics=("parallel",)),
  )(h, agg2, deg2, w1h, w1m, b1, w2, b2)


@jax.jit
def kernel(h, edge_index, W1, b1, W2, b2):
  src = edge_index[0]
  dst = edge_index[1]
  zagg = jnp.zeros((NP, D), jnp.float32)
  zdeg = jnp.zeros((NP, DEGW), jnp.float32)
  agg2, deg2 = _sc_aggregate()(src, dst, h, zagg, zdeg)
  return _tc_mlp(h, agg2, deg2, W1[:, :D], W1[:, D:],
                 b1.reshape(1, H), W2, b2.reshape(1, D))


# speedup vs baseline: 1.2490x; 1.0653x over previous
import functools

import jax
import jax.numpy as jnp
from jax import lax
from jax.experimental import pallas as pl
from jax.experimental.pallas import tpu as pltpu
from jax.experimental.pallas import tpu_sc as plsc

N = 10000
E = 320000
D = 128
H = 128

NC = 2
NS = 16
NW = NC * NS
EPW = E // NW
K = 80
CHUNKS = EPW // K
NP = 10240
RPT = NP // NS
DEGW = 16


def _sc_body(src_hbm, dst_hbm, h_hbm, zagg_hbm, zdeg_hbm,
             agg_out, deg_out,
             si0, si1, si2, si3, di0, di1, di2, di3, rows0, rows1,
             ones_v, agg_s, deg_s,
             isem0, isem1, isem2, isem3, gsem0, gsem1, ssem):
  cid = lax.axis_index("c")
  sid = lax.axis_index("s")
  wid = sid * NC + cid
  ebase = pl.multiple_of(wid * EPW, 8)
  rbase = pl.multiple_of(sid * RPT, 8)

  pltpu.sync_copy(zagg_hbm.at[pl.ds(rbase, RPT)], agg_s.at[pl.ds(rbase, RPT)])
  pltpu.sync_copy(zdeg_hbm.at[pl.ds(rbase, RPT)], deg_s.at[pl.ds(rbase, RPT)])
  for r in range(K):
    ones_v[r, :] = jnp.full((DEGW,), 1.0, jnp.float32)
  plsc.subcore_barrier()

  sis = (si0, si1, si2, si3)
  dis = (di0, di1, di2, di3)
  rows = (rows0, rows1)
  isems = (isem0, isem1, isem2, isem3)
  gsems = (gsem0, gsem1)

  def issue_idx(g, s):
    base = pl.multiple_of(ebase + g * K, 8)
    pltpu.async_copy(src_hbm.at[pl.ds(base, K)], sis[s], isems[s])
    pltpu.async_copy(dst_hbm.at[pl.ds(base, K)], dis[s], isems[s])

  def issue_gather(g, s, r):
    base = pl.multiple_of(ebase + g * K, 8)
    pltpu.make_async_copy(src_hbm.at[pl.ds(base, K)], sis[s], isems[s]).wait()
    pltpu.make_async_copy(dst_hbm.at[pl.ds(base, K)], dis[s], isems[s]).wait()
    pltpu.async_copy(h_hbm.at[sis[s]], rows[r], gsems[r])

  def finish(g, s, r):
    pltpu.make_async_copy(h_hbm.at[sis[s]], rows[r], gsems[r]).wait()
    pltpu.async_copy(rows[r], agg_s.at[dis[s]], ssem, add=True)
    pltpu.async_copy(ones_v, deg_s.at[dis[s]], ssem, add=True)
    pltpu.make_async_copy(rows[r], agg_s.at[dis[s]], ssem).wait()
    pltpu.make_async_copy(ones_v, deg_s.at[dis[s]], ssem).wait()

  issue_idx(0, 0)
  issue_idx(1, 1)
  issue_idx(2, 2)
  issue_idx(3, 3)
  issue_gather(0, 0, 0)

  @pl.loop(0, CHUNKS - 7, step=4)
  def _(i):
    issue_gather(i + 1, 1, 1)
    finish(i, 0, 0)
    issue_idx(i + 4, 0)
    issue_gather(i + 2, 2, 0)
    finish(i + 1, 1, 1)
    issue_idx(i + 5, 1)
    issue_gather(i + 3, 3, 1)
    finish(i + 2, 2, 0)
    issue_idx(i + 6, 2)
    issue_gather(i + 4, 0, 0)
    finish(i + 3, 3, 1)
    issue_idx(i + 7, 3)

  issue_gather(CHUNKS - 4, 1, 1)
  finish(CHUNKS - 5, 0, 0)
  issue_idx(CHUNKS - 1, 0)
  issue_gather(CHUNKS - 3, 2, 0)
  finish(CHUNKS - 4, 1, 1)
  issue_gather(CHUNKS - 2, 3, 1)
  finish(CHUNKS - 3, 2, 0)
  issue_gather(CHUNKS - 1, 0, 0)
  finish(CHUNKS - 2, 3, 1)
  finish(CHUNKS - 1, 0, 0)

  plsc.subcore_barrier()
  pltpu.sync_copy(agg_s.at[pl.ds(rbase, RPT)],
                  agg_out.at[cid, pl.ds(rbase, RPT)])
  pltpu.sync_copy(deg_s.at[pl.ds(rbase, RPT)],
                  deg_out.at[cid, pl.ds(rbase, RPT)])


@functools.lru_cache(maxsize=1)
def _sc_aggregate():
  return pl.kernel(
    _sc_body,
    out_type=[
        jax.ShapeDtypeStruct((NC, NP, D), jnp.float32),
        jax.ShapeDtypeStruct((NC, NP, DEGW), jnp.float32),
    ],
    mesh=plsc.VectorSubcoreMesh(
        core_axis_name="c", subcore_axis_name="s",
        num_cores=NC, num_subcores=NS),
    scratch_types=(
        [pltpu.VMEM((K,), jnp.int32)] * 8
        + [pltpu.VMEM((K, D), jnp.float32)] * 2
        + [
            pltpu.VMEM((K, DEGW), jnp.float32),
            pltpu.VMEM_SHARED((NP, D), jnp.float32),
            pltpu.VMEM_SHARED((NP, DEGW), jnp.float32),
        ]
        + [pltpu.SemaphoreType.DMA] * 7
    ),
    compiler_params=pltpu.CompilerParams(use_tc_tiling_on_sc=False),
  )


BN = 2000


def _tc_body(h_ref, agg_ref, deg_ref, w1h_ref, w1m_ref, b1_ref, w2_ref,
             b2_ref, out_ref):
  hh = h_ref[...]
  agg = agg_ref[0] + agg_ref[1]
  deg = (deg_ref[0] + deg_ref[1])[:, 0:1]
  mean = agg / jnp.maximum(deg, 1.0)
  mean = jnp.where(deg == 0.0, hh, mean)
  x = (lax.dot_general(hh, w1h_ref[...], (((1,), (1,)), ((), ())),
                       preferred_element_type=jnp.float32)
       + lax.dot_general(mean, w1m_ref[...], (((1,), (1,)), ((), ())),
                         preferred_element_type=jnp.float32)
       + b1_ref[...])
  x = 0.5 * x * (1.0 + lax.erf(x * (2.0 ** -0.5)))
  out_ref[...] = (lax.dot_general(x, w2_ref[...], (((1,), (1,)), ((), ())),
                                  preferred_element_type=jnp.float32)
                  + b2_ref[...])


def _tc_mlp(h, agg2, deg2, w1h, w1m, b1, w2, b2):
  return pl.pallas_call(
      _tc_body,
      grid=(N // BN,),
      in_specs=[
          pl.BlockSpec((BN, D), lambda i: (i, 0)),
          pl.BlockSpec((NC, BN, D), lambda i: (0, i, 0)),
          pl.BlockSpec((NC, BN, DEGW), lambda i: (0, i, 0)),
          pl.BlockSpec((H, D), lambda i: (0, 0)),
          pl.BlockSpec((H, D), lambda i: (0, 0)),
          pl.BlockSpec((1, H), lambda i: (0, 0)),
          pl.BlockSpec((D, H), lambda i: (0, 0)),
          pl.BlockSpec((1, D), lambda i: (0, 0)),
      ],
      out_specs=pl.BlockSpec((BN, D), lambda i: (i, 0)),
      out_shape=jax.ShapeDtypeStruct((N, D), jnp.float32),
      compiler_params=pltpu.CompilerParams(
          dimension_semantics=("parallel",)),
  )(h, agg2, deg2, w1h, w1m, b1, w2, b2)


@jax.jit
def kernel(h, edge_index, W1, b1, W2, b2):
  src = edge_index[0]
  dst = edge_index[1]
  zagg = jnp.zeros((NP, D), jnp.float32)
  zdeg = jnp.zeros((NP, DEGW), jnp.float32)
  agg2, deg2 = _sc_aggregate()(src, dst, h, zagg, zdeg)
  return _tc_mlp(h, agg2, deg2, W1[:, :D], W1[:, D:],
                 b1.reshape(1, H), W2, b2.reshape(1, D))


# speedup vs baseline: 1.2515x; 1.0020x over previous
import functools

import jax
import jax.numpy as jnp
from jax import lax
from jax.experimental import pallas as pl
from jax.experimental.pallas import tpu as pltpu
from jax.experimental.pallas import tpu_sc as plsc

N = 10000
E = 320000
D = 128
H = 128

NC = 2
NS = 16
NW = NC * NS
EPW = E // NW
K = 80
CHUNKS = EPW // K
NP = 10240
RPT = NP // NS
DEGW = 16


def _sc_body(src_hbm, dst_hbm, h_hbm, zagg_hbm, zdeg_hbm,
             agg_out, deg_out,
             si0, si1, si2, si3, di0, di1, di2, di3, rows0, rows1,
             ones_v, agg_s, deg_s,
             isem0, isem1, isem2, isem3, gsem0, gsem1, ssem):
  cid = lax.axis_index("c")
  sid = lax.axis_index("s")
  wid = sid * NC + cid
  ebase = pl.multiple_of(wid * EPW, 8)
  rbase = pl.multiple_of(sid * RPT, 8)

  pltpu.sync_copy(zagg_hbm.at[pl.ds(rbase, RPT)], agg_s.at[pl.ds(rbase, RPT)])
  pltpu.sync_copy(zdeg_hbm.at[pl.ds(rbase, RPT)], deg_s.at[pl.ds(rbase, RPT)])
  for r in range(K):
    ones_v[r, :] = jnp.full((DEGW,), 1.0, jnp.float32)
  plsc.subcore_barrier()

  sis = (si0, si1, si2, si3)
  dis = (di0, di1, di2, di3)
  rows = (rows0, rows1)
  isems = (isem0, isem1, isem2, isem3)
  gsems = (gsem0, gsem1)

  def issue_idx(g, s):
    base = pl.multiple_of(ebase + g * K, 8)
    pltpu.async_copy(src_hbm.at[pl.ds(base, K)], sis[s], isems[s])
    pltpu.async_copy(dst_hbm.at[pl.ds(base, K)], dis[s], isems[s])

  def issue_gather(g, s, r):
    base = pl.multiple_of(ebase + g * K, 8)
    pltpu.make_async_copy(src_hbm.at[pl.ds(base, K)], sis[s], isems[s]).wait()
    pltpu.make_async_copy(dst_hbm.at[pl.ds(base, K)], dis[s], isems[s]).wait()
    pltpu.async_copy(h_hbm.at[sis[s]], rows[r], gsems[r])

  def finish(g, s, r):
    pltpu.make_async_copy(h_hbm.at[sis[s]], rows[r], gsems[r]).wait()
    pltpu.async_copy(rows[r], agg_s.at[dis[s]], ssem, add=True)
    pltpu.async_copy(ones_v, deg_s.at[dis[s]], ssem, add=True)
    pltpu.make_async_copy(rows[r], agg_s.at[dis[s]], ssem).wait()
    pltpu.make_async_copy(ones_v, deg_s.at[dis[s]], ssem).wait()

  issue_idx(0, 0)
  issue_idx(1, 1)
  issue_idx(2, 2)
  issue_idx(3, 3)
  issue_gather(0, 0, 0)

  @pl.loop(0, CHUNKS - 7, step=4)
  def _(i):
    issue_gather(i + 1, 1, 1)
    finish(i, 0, 0)
    issue_idx(i + 4, 0)
    issue_gather(i + 2, 2, 0)
    finish(i + 1, 1, 1)
    issue_idx(i + 5, 1)
    issue_gather(i + 3, 3, 1)
    finish(i + 2, 2, 0)
    issue_idx(i + 6, 2)
    issue_gather(i + 4, 0, 0)
    finish(i + 3, 3, 1)
    issue_idx(i + 7, 3)

  issue_gather(CHUNKS - 4, 1, 1)
  finish(CHUNKS - 5, 0, 0)
  issue_idx(CHUNKS - 1, 0)
  issue_gather(CHUNKS - 3, 2, 0)
  finish(CHUNKS - 4, 1, 1)
  issue_gather(CHUNKS - 2, 3, 1)
  finish(CHUNKS - 3, 2, 0)
  issue_gather(CHUNKS - 1, 0, 0)
  finish(CHUNKS - 2, 3, 1)
  finish(CHUNKS - 1, 0, 0)

  plsc.subcore_barrier()
  pltpu.sync_copy(agg_s.at[pl.ds(rbase, RPT)],
                  agg_out.at[cid, pl.ds(rbase, RPT)])
  pltpu.sync_copy(deg_s.at[pl.ds(rbase, RPT)],
                  deg_out.at[cid, pl.ds(rbase, RPT)])


@functools.lru_cache(maxsize=1)
def _sc_aggregate():
  return pl.kernel(
    _sc_body,
    out_type=[
        jax.ShapeDtypeStruct((NC, NP, D), jnp.float32),
        jax.ShapeDtypeStruct((NC, NP, DEGW), jnp.float32),
    ],
    mesh=plsc.VectorSubcoreMesh(
        core_axis_name="c", subcore_axis_name="s",
        num_cores=NC, num_subcores=NS),
    scratch_types=(
        [pltpu.VMEM((K,), jnp.int32)] * 8
        + [pltpu.VMEM((K, D), jnp.float32)] * 2
        + [
            pltpu.VMEM((K, DEGW), jnp.float32),
            pltpu.VMEM_SHARED((NP, D), jnp.float32),
            pltpu.VMEM_SHARED((NP, DEGW), jnp.float32),
        ]
        + [pltpu.SemaphoreType.DMA] * 7
    ),
    compiler_params=pltpu.CompilerParams(use_tc_tiling_on_sc=False),
  )


BN = 5000


def _tc_body(h_ref, agg_ref, deg_ref, w1h_ref, w1m_ref, b1_ref, w2_ref,
             b2_ref, out_ref):
  hh = h_ref[...]
  agg = agg_ref[0] + agg_ref[1]
  deg = (deg_ref[0] + deg_ref[1])[:, 0:1]
  mean = agg / jnp.maximum(deg, 1.0)
  mean = jnp.where(deg == 0.0, hh, mean)
  x = (lax.dot_general(hh, w1h_ref[...], (((1,), (1,)), ((), ())),
                       preferred_element_type=jnp.float32)
       + lax.dot_general(mean, w1m_ref[...], (((1,), (1,)), ((), ())),
                         preferred_element_type=jnp.float32)
       + b1_ref[...])
  x = 0.5 * x * (1.0 + lax.erf(x * (2.0 ** -0.5)))
  out_ref[...] = (lax.dot_general(x, w2_ref[...], (((1,), (1,)), ((), ())),
                                  preferred_element_type=jnp.float32)
                  + b2_ref[...])


def _tc_mlp(h, agg2, deg2, w1h, w1m, b1, w2, b2):
  return pl.pallas_call(
      _tc_body,
      grid=(N // BN,),
      in_specs=[
          pl.BlockSpec((BN, D), lambda i: (i, 0)),
          pl.BlockSpec((NC, BN, D), lambda i: (0, i, 0)),
          pl.BlockSpec((NC, BN, DEGW), lambda i: (0, i, 0)),
          pl.BlockSpec((H, D), lambda i: (0, 0)),
          pl.BlockSpec((H, D), lambda i: (0, 0)),
          pl.BlockSpec((1, H), lambda i: (0, 0)),
          pl.BlockSpec((D, H), lambda i: (0, 0)),
          pl.BlockSpec((1, D), lambda i: (0, 0)),
      ],
      out_specs=pl.BlockSpec((BN, D), lambda i: (i, 0)),
      out_shape=jax.ShapeDtypeStruct((N, D), jnp.float32),
      compiler_params=pltpu.CompilerParams(
          dimension_semantics=("parallel",)),
  )(h, agg2, deg2, w1h, w1m, b1, w2, b2)


@jax.jit
def kernel(h, edge_index, W1, b1, W2, b2):
  src = edge_index[0]
  dst = edge_index[1]
  zagg = jnp.zeros((NP, D), jnp.float32)
  zdeg = jnp.zeros((NP, DEGW), jnp.float32)
  agg2, deg2 = _sc_aggregate()(src, dst, h, zagg, zdeg)
  return _tc_mlp(h, agg2, deg2, W1[:, :D], W1[:, D:],
                 b1.reshape(1, H), W2, b2.reshape(1, D))
